# baseline (speedup 1.0000x reference)
import functools

import jax
import jax.numpy as jnp
from jax import lax
from jax.experimental import pallas as pl
from jax.experimental.pallas import tpu as pltpu
from jax.experimental.pallas import tpu_sc as plsc

B = 1024
A = 32
F = 41024
H = 512
HL = 128
Q = H // HL

NC = 2
NS = 16
L = 16
NW = NC * NS
RPW = B // NW
NCHUNK = 2 * RPW
GPC = A * Q
TRASH = NS * NCHUNK


def _sc_embed_body(widx, bidx, tab4, out, allidx, gat, dstb, rows0, rows1,
                   stage, acc, sem0, sem1):
    cid = lax.axis_index("c")
    sid = lax.axis_index("s")
    w = cid * NS + sid
    b0 = w * RPW
    lane = lax.iota(jnp.int32, L)

    pltpu.sync_copy(widx.at[pl.ds(b0 * A, RPW * A)], allidx.at[pl.ds(0, RPW * A)])
    pltpu.sync_copy(bidx.at[pl.ds(b0 * A, RPW * A)], allidx.at[pl.ds(RPW * A, RPW * A)])

    def build(it, _):
        g = it >> 1
        s = it & 1
        local_b = g * L + lane
        base = (s * RPW + local_b) * A
        ts = [plsc.load_gather(allidx, [base + p]) for p in range(A)]
        ci = local_b * 2 + s
        accrow = sid * NCHUNK + ci
        dd = [accrow]
        for i in range(1, A):
            m = ts[i] == ts[0]
            for j in range(1, i):
                m = m | (ts[i] == ts[j])
            dd.append(jnp.where(m, TRASH, accrow))
        for a in range(A):
            t4 = ts[a] * Q
            d4 = dd[a] * Q
            for q in range(Q):
                col = jnp.full((L,), a * Q + q, jnp.int32)
                plsc.store_scatter(gat, [ci, col], t4 + q)
                plsc.store_scatter(dstb, [ci, col], d4 + q)
        return 0

    lax.fori_loop(0, 4, build, 0, unroll=False)

    zero = jnp.zeros((L,), jnp.float32)

    def zloop(r, _):
        for c in range(HL // L):
            stage[r, pl.ds(c * L, L)] = zero
        return 0

    lax.fori_loop(0, NCHUNK * Q, zloop, 0, unroll=False)
    pltpu.sync_copy(stage, acc.at[pl.ds(sid * NCHUNK * Q, NCHUNK * Q)])

    pltpu.async_copy(tab4.at[gat.at[0]], rows0, sem0)

    def chunk(i, _):
        ci = i * 2
        pltpu.async_copy(tab4.at[gat.at[ci + 1]], rows1, sem1)
        pltpu.make_async_copy(tab4.at[gat.at[ci]], rows0, sem0).wait()
        pltpu.sync_copy(rows0, acc.at[dstb.at[ci]], add=True)

        @pl.when(i < NCHUNK // 2 - 1)
        def _():
            pltpu.async_copy(tab4.at[gat.at[ci + 2]], rows0, sem0)

        pltpu.make_async_copy(tab4.at[gat.at[ci + 1]], rows1, sem1).wait()
        pltpu.sync_copy(rows1, acc.at[dstb.at[ci + 1]], add=True)
        return 0

    lax.fori_loop(0, NCHUNK // 2, chunk, 0, unroll=False)

    pltpu.sync_copy(acc.at[pl.ds(sid * NCHUNK * Q, NCHUNK * Q)], stage)

    def cloop(r, _):
        for c in range(HL // L):
            v = stage[r, pl.ds(c * L, L)]
            stage[r, pl.ds(c * L, L)] = jnp.minimum(jnp.maximum(v, -1.0), 1.0)
        return 0

    lax.fori_loop(0, NCHUNK * Q, cloop, 0, unroll=False)
    pltpu.sync_copy(stage, out.at[pl.ds(w * NCHUNK * Q, NCHUNK * Q)])


_sc_embed = functools.partial(
    pl.kernel,
    out_type=jax.ShapeDtypeStruct((2 * B * Q, HL), jnp.float32),
    mesh=plsc.VectorSubcoreMesh(
        core_axis_name="c", subcore_axis_name="s", num_cores=NC, num_subcores=NS
    ),
    compiler_params=pltpu.CompilerParams(needs_layout_passes=False),
    scratch_types=[
        pltpu.VMEM((2 * RPW * A,), jnp.int32),
        pltpu.VMEM((NCHUNK, GPC), jnp.int32),
        pltpu.VMEM((NCHUNK, GPC), jnp.int32),
        pltpu.VMEM((GPC, HL), jnp.float32),
        pltpu.VMEM((GPC, HL), jnp.float32),
        pltpu.VMEM((NCHUNK * Q, HL), jnp.float32),
        pltpu.VMEM_SHARED((NS * NCHUNK * Q + Q, HL), jnp.float32),
        pltpu.SemaphoreType.DMA,
        pltpu.SemaphoreType.DMA,
    ],
)(_sc_embed_body)


def _tr_body(x_ref, o_ref):
    o_ref[...] = x_ref[...].T


def _transpose_table(W_ft):
    CW = 128
    return pl.pallas_call(
        _tr_body,
        grid=(pl.cdiv(F, CW),),
        in_specs=[pl.BlockSpec((H, CW), lambda i: (0, i))],
        out_specs=pl.BlockSpec((CW, H), lambda i: (i, 0)),
        out_shape=jax.ShapeDtypeStruct((F, H), jnp.float32),
    )(W_ft)


def _mlp_body(x_ref, w1, b1, w2, b2, w3, b3, wo, o_ref):
    cdims = (((1,), (1,)), ((), ()))
    h = lax.dot_general(x_ref[...], w1[...], cdims,
                        preferred_element_type=jnp.float32)
    h = jnp.maximum(h + b1[...], 0.0)
    h = lax.dot_general(h, w2[...], cdims, preferred_element_type=jnp.float32)
    h = jnp.maximum(h + b2[...], 0.0)
    h = lax.dot_general(h, w3[...], cdims, preferred_element_type=jnp.float32)
    h = jnp.maximum(h + b3[...], 0.0)
    o_ref[...] = lax.dot_general(h, wo[...], cdims,
                                 preferred_element_type=jnp.float32)


def _mlp(x, W1, b1, W2, b2, W3, b3, W_out):
    BM = 512
    full = lambda i: (0, 0)
    return pl.pallas_call(
        _mlp_body,
        grid=(B // BM,),
        in_specs=[
            pl.BlockSpec((BM, 2 * H), lambda i: (i, 0)),
            pl.BlockSpec((H, 2 * H), full),
            pl.BlockSpec((1, H), full),
            pl.BlockSpec((H // 2, H), full),
            pl.BlockSpec((1, H // 2), full),
            pl.BlockSpec((H // 4, H // 2), full),
            pl.BlockSpec((1, H // 4), full),
            pl.BlockSpec((1, H // 4), full),
        ],
        out_specs=pl.BlockSpec((BM, 1), lambda i: (i, 0)),
        out_shape=jax.ShapeDtypeStruct((B, 1), jnp.float32),
    )(x, W1, b1.reshape(1, H), W2, b2.reshape(1, H // 2),
      W3, b3.reshape(1, H // 4), W_out)


def kernel(white_indices, black_indices, W_ft, W1, b1, W2, b2, W3, b3, W_out, b_out):
    tab4 = _transpose_table(W_ft).reshape(-1, HL)
    ft = _sc_embed(white_indices.astype(jnp.int32).reshape(-1),
                   black_indices.astype(jnp.int32).reshape(-1), tab4)
    x = ft.reshape(B, 2 * H)
    out = _mlp(x, W1, b1, W2, b2, W3, b3, W_out)
    return out[:, 0] + b_out


# speedup vs baseline: 1.9219x; 1.9219x over previous
import functools

import jax
import jax.numpy as jnp
from jax import lax
from jax.experimental import pallas as pl
from jax.experimental.pallas import tpu as pltpu
from jax.experimental.pallas import tpu_sc as plsc

B = 1024
A = 32
F = 41024
H = 512
HL = 128

NC = 2
NS = 16
L = 16
NW = NC * NS
RPW = B // NW
NCHUNK = 2 * RPW
TRASH = NS * NCHUNK


def _make_sc_embed(Q):
    GPC = A * Q

    def body(widx, bidx, tab, out, allidx, gat, dstb, rows0, rows1,
             stage, acc, sem0, sem1):
        cid = lax.axis_index("c")
        sid = lax.axis_index("s")
        w = cid * NS + sid
        b0 = w * RPW
        lane = lax.iota(jnp.int32, L)

        pltpu.sync_copy(widx.at[pl.ds(b0 * A, RPW * A)],
                        allidx.at[pl.ds(0, RPW * A)])
        pltpu.sync_copy(bidx.at[pl.ds(b0 * A, RPW * A)],
                        allidx.at[pl.ds(RPW * A, RPW * A)])

        def build(it, _):
            g = it >> 1
            s = it & 1
            local_b = g * L + lane
            base = (s * RPW + local_b) * A
            ts = [plsc.load_gather(allidx, [base + p]) for p in range(A)]
            ci = local_b * 2 + s
            accrow = sid * NCHUNK + ci
            dd = [accrow]
            for i in range(1, A):
                m = ts[i] == ts[0]
                for j in range(1, i):
                    m = m | (ts[i] == ts[j])
                dd.append(jnp.where(m, TRASH, accrow))
            for a in range(A):
                tq = ts[a] * Q
                dq = dd[a] * Q
                for q in range(Q):
                    col = jnp.full((L,), a * Q + q, jnp.int32)
                    plsc.store_scatter(gat, [ci, col], tq + q)
                    plsc.store_scatter(dstb, [ci, col], dq + q)
            return 0

        lax.fori_loop(0, 4, build, 0, unroll=False)

        zero = jnp.zeros((L,), jnp.float32)

        def zloop(r, _):
            for c in range(HL // L):
                stage[r, pl.ds(c * L, L)] = zero
            return 0

        lax.fori_loop(0, NCHUNK * Q, zloop, 0, unroll=False)
        pltpu.sync_copy(stage, acc.at[pl.ds(sid * NCHUNK * Q, NCHUNK * Q)])

        pltpu.async_copy(tab.at[gat.at[0]], rows0, sem0)

        def chunk(i, _):
            ci = i * 2
            pltpu.async_copy(tab.at[gat.at[ci + 1]], rows1, sem1)
            pltpu.make_async_copy(tab.at[gat.at[ci]], rows0, sem0).wait()
            pltpu.sync_copy(rows0, acc.at[dstb.at[ci]], add=True)

            @pl.when(i < NCHUNK // 2 - 1)
            def _():
                pltpu.async_copy(tab.at[gat.at[ci + 2]], rows0, sem0)

            pltpu.make_async_copy(tab.at[gat.at[ci + 1]], rows1, sem1).wait()
            pltpu.sync_copy(rows1, acc.at[dstb.at[ci + 1]], add=True)
            return 0

        lax.fori_loop(0, NCHUNK // 2, chunk, 0, unroll=False)

        pltpu.sync_copy(acc.at[pl.ds(sid * NCHUNK * Q, NCHUNK * Q)], stage)

        def cloop(r, _):
            for c in range(HL // L):
                v = stage[r, pl.ds(c * L, L)]
                stage[r, pl.ds(c * L, L)] = jnp.minimum(jnp.maximum(v, -1.0), 1.0)
            return 0

        lax.fori_loop(0, NCHUNK * Q, cloop, 0, unroll=False)
        pltpu.sync_copy(stage, out.at[pl.ds(w * NCHUNK * Q, NCHUNK * Q)])

    return functools.partial(
        pl.kernel,
        out_type=jax.ShapeDtypeStruct((2 * B * Q, HL), jnp.float32),
        mesh=plsc.VectorSubcoreMesh(
            core_axis_name="c", subcore_axis_name="s",
            num_cores=NC, num_subcores=NS,
        ),
        compiler_params=pltpu.CompilerParams(needs_layout_passes=False),
        scratch_types=[
            pltpu.VMEM((2 * RPW * A,), jnp.int32),
            pltpu.VMEM((NCHUNK, GPC), jnp.int32),
            pltpu.VMEM((NCHUNK, GPC), jnp.int32),
            pltpu.VMEM((GPC, HL), jnp.float32),
            pltpu.VMEM((GPC, HL), jnp.float32),
            pltpu.VMEM((NCHUNK * Q, HL), jnp.float32),
            pltpu.VMEM_SHARED((NS * NCHUNK * Q + Q, HL), jnp.float32),
            pltpu.SemaphoreType.DMA,
            pltpu.SemaphoreType.DMA,
        ],
    )(body)


_sc_embed_half = _make_sc_embed(H // (2 * HL))


def _mlp_body(x_ref, w1, b1, w2, b2, w3, b3, wo, o_ref):
    cdims = (((1,), (1,)), ((), ()))
    h = lax.dot_general(x_ref[...], w1[...], cdims,
                        preferred_element_type=jnp.float32)
    h = jnp.maximum(h + b1[...], 0.0)
    h = lax.dot_general(h, w2[...], cdims, preferred_element_type=jnp.float32)
    h = jnp.maximum(h + b2[...], 0.0)
    h = lax.dot_general(h, w3[...], cdims, preferred_element_type=jnp.float32)
    h = jnp.maximum(h + b3[...], 0.0)
    o_ref[...] = lax.dot_general(h, wo[...], cdims,
                                 preferred_element_type=jnp.float32)


def _mlp(x, W1, b1, W2, b2, W3, b3, W_out):
    BM = 512
    full = lambda i: (0, 0)
    return pl.pallas_call(
        _mlp_body,
        grid=(B // BM,),
        in_specs=[
            pl.BlockSpec((BM, 2 * H), lambda i: (i, 0)),
            pl.BlockSpec((H, 2 * H), full),
            pl.BlockSpec((1, H), full),
            pl.BlockSpec((H // 2, H), full),
            pl.BlockSpec((1, H // 2), full),
            pl.BlockSpec((H // 4, H // 2), full),
            pl.BlockSpec((1, H // 4), full),
            pl.BlockSpec((1, H // 4), full),
        ],
        out_specs=pl.BlockSpec((BM, 1), lambda i: (i, 0)),
        out_shape=jax.ShapeDtypeStruct((B, 1), jnp.float32),
    )(x, W1, b1.reshape(1, H), W2, b2.reshape(1, H // 2),
      W3, b3.reshape(1, H // 4), W_out)


def kernel(white_indices, black_indices, W_ft, W1, b1, W2, b2, W3, b3, W_out, b_out):
    wi = white_indices.astype(jnp.int32).reshape(-1)
    bi = black_indices.astype(jnp.int32).reshape(-1)
    tabA = W_ft[:H // 2].T.reshape(-1, HL)
    ftA = _sc_embed_half(wi, bi, tabA)
    tabB = W_ft[H // 2:].T.reshape(-1, HL)
    ftB = _sc_embed_half(wi, bi, tabB)
    x = jnp.concatenate([ftA.reshape(B, 2, 2, HL),
                         ftB.reshape(B, 2, 2, HL)], axis=2).reshape(B, 2 * H)
    out = _mlp(x, W1, b1, W2, b2, W3, b3, W_out)
    return out[:, 0] + b_out


# speedup vs baseline: 2.3936x; 1.2454x over previous
import functools

import jax
import jax.numpy as jnp
from jax import lax
from jax.experimental import pallas as pl
from jax.experimental.pallas import tpu as pltpu
from jax.experimental.pallas import tpu_sc as plsc

B = 1024
A = 32
F = 41024
H = 512
HL = 128
Q = H // HL

NC = 2
NS = 16
L = 16
NW = NC * NS
RPW = B // NW
NCHUNK = 2 * RPW
GPC = A * Q
TRASH = NS * NCHUNK


def _sc_embed_body(widx, bidx, tab4, out, allidx, gat, dstb, rows0, rows1,
                   stage, acc, sem0, sem1):
    cid = lax.axis_index("c")
    sid = lax.axis_index("s")
    w = cid * NS + sid
    b0 = w * RPW
    lane = lax.iota(jnp.int32, L)

    pltpu.sync_copy(widx.at[pl.ds(b0 * A, RPW * A)], allidx.at[pl.ds(0, RPW * A)])
    pltpu.sync_copy(bidx.at[pl.ds(b0 * A, RPW * A)], allidx.at[pl.ds(RPW * A, RPW * A)])

    def build(it, _):
        g = it >> 1
        s = it & 1
        local_b = g * L + lane
        base = (s * RPW + local_b) * A
        ts = [plsc.load_gather(allidx, [base + p]) for p in range(A)]
        ci = local_b * 2 + s
        accrow = sid * NCHUNK + ci
        dd = [accrow]
        for i in range(1, A):
            m = ts[i] == ts[0]
            for j in range(1, i):
                m = m | (ts[i] == ts[j])
            dd.append(jnp.where(m, TRASH, accrow))
        for a in range(A):
            t4 = ts[a] * Q
            d4 = dd[a] * Q
            for q in range(Q):
                col = jnp.full((L,), a * Q + q, jnp.int32)
                plsc.store_scatter(gat, [ci, col], t4 + q)
                plsc.store_scatter(dstb, [ci, col], d4 + q)
        return 0

    lax.fori_loop(0, 4, build, 0, unroll=False)

    zero = jnp.zeros((L,), jnp.float32)

    def zloop(r, _):
        for c in range(HL // L):
            stage[r, pl.ds(c * L, L)] = zero
        return 0

    lax.fori_loop(0, NCHUNK * Q, zloop, 0, unroll=False)
    pltpu.sync_copy(stage, acc.at[pl.ds(sid * NCHUNK * Q, NCHUNK * Q)])

    pltpu.async_copy(tab4.at[gat.at[0]], rows0, sem0)

    def chunk(i, _):
        ci = i * 2
        pltpu.async_copy(tab4.at[gat.at[ci + 1]], rows1, sem1)
        pltpu.make_async_copy(tab4.at[gat.at[ci]], rows0, sem0).wait()
        pltpu.sync_copy(rows0, acc.at[dstb.at[ci]], add=True)

        @pl.when(i < NCHUNK // 2 - 1)
        def _():
            pltpu.async_copy(tab4.at[gat.at[ci + 2]], rows0, sem0)

        pltpu.make_async_copy(tab4.at[gat.at[ci + 1]], rows1, sem1).wait()
        pltpu.sync_copy(rows1, acc.at[dstb.at[ci + 1]], add=True)
        return 0

    lax.fori_loop(0, NCHUNK // 2, chunk, 0, unroll=False)

    pltpu.sync_copy(acc.at[pl.ds(sid * NCHUNK * Q, NCHUNK * Q)], stage)

    def cloop(r, _):
        for c in range(HL // L):
            v = stage[r, pl.ds(c * L, L)]
            stage[r, pl.ds(c * L, L)] = jnp.minimum(jnp.maximum(v, -1.0), 1.0)
        return 0

    lax.fori_loop(0, NCHUNK * Q, cloop, 0, unroll=False)
    pltpu.sync_copy(stage, out.at[pl.ds(w * NCHUNK * Q, NCHUNK * Q)])


_sc_embed = functools.partial(
    pl.kernel,
    out_type=jax.ShapeDtypeStruct((2 * B * Q, HL), jnp.float32),
    mesh=plsc.VectorSubcoreMesh(
        core_axis_name="c", subcore_axis_name="s", num_cores=NC, num_subcores=NS
    ),
    compiler_params=pltpu.CompilerParams(needs_layout_passes=False),
    scratch_types=[
        pltpu.VMEM((2 * RPW * A,), jnp.int32),
        pltpu.VMEM((NCHUNK, GPC), jnp.int32),
        pltpu.VMEM((NCHUNK, GPC), jnp.int32),
        pltpu.VMEM((GPC, HL), jnp.float32),
        pltpu.VMEM((GPC, HL), jnp.float32),
        pltpu.VMEM((NCHUNK * Q, HL), jnp.float32),
        pltpu.VMEM_SHARED((NS * NCHUNK * Q + Q, HL), jnp.float32),
        pltpu.SemaphoreType.DMA,
        pltpu.SemaphoreType.DMA,
    ],
)(_sc_embed_body)


def _mlp_body(x_ref, w1, b1, w2, b2, w3, b3, wo, o_ref):
    cdims = (((1,), (1,)), ((), ()))
    h = lax.dot_general(x_ref[...], w1[...], cdims,
                        preferred_element_type=jnp.float32)
    h = jnp.maximum(h + b1[...], 0.0)
    h = lax.dot_general(h, w2[...], cdims, preferred_element_type=jnp.float32)
    h = jnp.maximum(h + b2[...], 0.0)
    h = lax.dot_general(h, w3[...], cdims, preferred_element_type=jnp.float32)
    h = jnp.maximum(h + b3[...], 0.0)
    o_ref[...] = lax.dot_general(h, wo[...], cdims,
                                 preferred_element_type=jnp.float32)


def _mlp(x, W1, b1, W2, b2, W3, b3, W_out):
    BM = 1024
    full = lambda i: (0, 0)
    return pl.pallas_call(
        _mlp_body,
        grid=(B // BM,),
        in_specs=[
            pl.BlockSpec((BM, 2 * H), lambda i: (i, 0)),
            pl.BlockSpec((H, 2 * H), full),
            pl.BlockSpec((1, H), full),
            pl.BlockSpec((H // 2, H), full),
            pl.BlockSpec((1, H // 2), full),
            pl.BlockSpec((H // 4, H // 2), full),
            pl.BlockSpec((1, H // 4), full),
            pl.BlockSpec((1, H // 4), full),
        ],
        out_specs=pl.BlockSpec((BM, 1), lambda i: (i, 0)),
        out_shape=jax.ShapeDtypeStruct((B, 1), jnp.float32),
    )(x, W1, b1.reshape(1, H), W2, b2.reshape(1, H // 2),
      W3, b3.reshape(1, H // 4), W_out)


def kernel(white_indices, black_indices, W_ft, W1, b1, W2, b2, W3, b3, W_out, b_out):
    tab4 = W_ft.T.reshape(-1, HL)
    ft = _sc_embed(white_indices.astype(jnp.int32).reshape(-1),
                   black_indices.astype(jnp.int32).reshape(-1), tab4)
    x = ft.reshape(B, 2 * H)
    out = _mlp(x, W1, b1, W2, b2, W3, b3, W_out)
    return out[:, 0] + b_out


# speedup vs baseline: 2.5658x; 1.0720x over previous
import functools

import jax
import jax.numpy as jnp
from jax import lax
from jax.experimental import pallas as pl
from jax.experimental.pallas import tpu as pltpu
from jax.experimental.pallas import tpu_sc as plsc

B = 1024
A = 32
F = 41024
H = 512
HL = 128
Q = H // HL

NC = 2
NS = 16
L = 16
NW = NC * NS
RPW = B // NW
NCHUNK = 2 * RPW
GPC = A * Q
TRASH = NS * NCHUNK


NBUF = 4
SPASS = 4


def _sc_embed_body(widx, bidx, tab4, out, allidx, gat, dstb,
                   rows0, rows1, rows2, rows3,
                   stage, acc, sem0, sem1, sem2, sem3):
    cid = lax.axis_index("c")
    sid = lax.axis_index("s")
    w = cid * NS + sid
    b0 = w * RPW
    lane = lax.iota(jnp.int32, L)

    pltpu.sync_copy(widx.at[pl.ds(b0 * A, RPW * A)], allidx.at[pl.ds(0, RPW * A)])
    pltpu.sync_copy(bidx.at[pl.ds(b0 * A, RPW * A)], allidx.at[pl.ds(RPW * A, RPW * A)])

    def build(it, _):
        g = it >> 1
        s = it & 1
        local_b = g * L + lane
        base = (s * RPW + local_b) * A
        ts = [plsc.load_gather(allidx, [base + p]) for p in range(A)]
        ci = local_b * 2 + s
        accrow = sid * NCHUNK + ci
        dd = [accrow]
        for i in range(1, A):
            m = ts[i] == ts[0]
            for j in range(1, i):
                m = m | (ts[i] == ts[j])
            dd.append(jnp.where(m, TRASH, accrow))
        for a in range(A):
            t4 = ts[a] * Q
            d4 = dd[a] * Q
            for q in range(Q):
                col = jnp.full((L,), a * Q + q, jnp.int32)
                plsc.store_scatter(gat, [ci, col], t4 + q)
                plsc.store_scatter(dstb, [ci, col], d4 + q)
        return 0

    lax.fori_loop(0, 4, build, 0, unroll=False)

    rows = [rows0, rows1, rows2, rows3]
    sems = [sem0, sem1, sem2, sem3]
    SROWS = NCHUNK * Q // SPASS

    zero = jnp.zeros((L,), jnp.float32)

    def zloop(r, _):
        for c in range(HL // L):
            stage[r, pl.ds(c * L, L)] = zero
        return 0

    lax.fori_loop(0, SROWS, zloop, 0, unroll=False)
    for p in range(SPASS):
        pltpu.sync_copy(stage,
                        acc.at[pl.ds(sid * NCHUNK * Q + p * SROWS, SROWS)])

    for k in range(NBUF):
        pltpu.async_copy(tab4.at[gat.at[k]], rows[k], sems[k])

    def chunk(i, _):
        ci = i * NBUF
        for k in range(NBUF):
            pltpu.make_async_copy(tab4.at[gat.at[ci + k]], rows[k],
                                  sems[k]).wait()
            pltpu.sync_copy(rows[k], acc.at[dstb.at[ci + k]], add=True)

            @pl.when(i < NCHUNK // NBUF - 1)
            def _():
                pltpu.async_copy(tab4.at[gat.at[ci + NBUF + k]], rows[k],
                                 sems[k])
        return 0

    lax.fori_loop(0, NCHUNK // NBUF, chunk, 0, unroll=False)

    def cloop(r, _):
        for c in range(HL // L):
            v = stage[r, pl.ds(c * L, L)]
            stage[r, pl.ds(c * L, L)] = jnp.minimum(jnp.maximum(v, -1.0), 1.0)
        return 0

    for p in range(SPASS):
        pltpu.sync_copy(
            acc.at[pl.ds(sid * NCHUNK * Q + p * SROWS, SROWS)], stage)
        lax.fori_loop(0, SROWS, cloop, 0, unroll=False)
        pltpu.sync_copy(
            stage, out.at[pl.ds(w * NCHUNK * Q + p * SROWS, SROWS)])


_sc_embed = functools.partial(
    pl.kernel,
    out_type=jax.ShapeDtypeStruct((2 * B * Q, HL), jnp.float32),
    mesh=plsc.VectorSubcoreMesh(
        core_axis_name="c", subcore_axis_name="s", num_cores=NC, num_subcores=NS
    ),
    compiler_params=pltpu.CompilerParams(needs_layout_passes=False),
    scratch_types=[
        pltpu.VMEM((2 * RPW * A,), jnp.int32),
        pltpu.VMEM((NCHUNK, GPC), jnp.int32),
        pltpu.VMEM((NCHUNK, GPC), jnp.int32),
        pltpu.VMEM((GPC, HL), jnp.float32),
        pltpu.VMEM((GPC, HL), jnp.float32),
        pltpu.VMEM((GPC, HL), jnp.float32),
        pltpu.VMEM((GPC, HL), jnp.float32),
        pltpu.VMEM((NCHUNK * Q // 4, HL), jnp.float32),
        pltpu.VMEM_SHARED((NS * NCHUNK * Q + Q, HL), jnp.float32),
        pltpu.SemaphoreType.DMA,
        pltpu.SemaphoreType.DMA,
        pltpu.SemaphoreType.DMA,
        pltpu.SemaphoreType.DMA,
    ],
)(_sc_embed_body)


def _mlp_body(x_ref, w1, b1, w2, b2, w3, b3, wo, o_ref):
    cdims = (((1,), (1,)), ((), ()))
    h = lax.dot_general(x_ref[...], w1[...], cdims,
                        preferred_element_type=jnp.float32)
    h = jnp.maximum(h + b1[...], 0.0)
    h = lax.dot_general(h, w2[...], cdims, preferred_element_type=jnp.float32)
    h = jnp.maximum(h + b2[...], 0.0)
    h = lax.dot_general(h, w3[...], cdims, preferred_element_type=jnp.float32)
    h = jnp.maximum(h + b3[...], 0.0)
    o_ref[...] = lax.dot_general(h, wo[...], cdims,
                                 preferred_element_type=jnp.float32)


def _mlp(x, W1, b1, W2, b2, W3, b3, W_out):
    BM = 1024
    full = lambda i: (0, 0)
    return pl.pallas_call(
        _mlp_body,
        grid=(B // BM,),
        in_specs=[
            pl.BlockSpec((BM, 2 * H), lambda i: (i, 0)),
            pl.BlockSpec((H, 2 * H), full),
            pl.BlockSpec((1, H), full),
            pl.BlockSpec((H // 2, H), full),
            pl.BlockSpec((1, H // 2), full),
            pl.BlockSpec((H // 4, H // 2), full),
            pl.BlockSpec((1, H // 4), full),
            pl.BlockSpec((1, H // 4), full),
        ],
        out_specs=pl.BlockSpec((BM, 1), lambda i: (i, 0)),
        out_shape=jax.ShapeDtypeStruct((B, 1), jnp.float32),
    )(x, W1, b1.reshape(1, H), W2, b2.reshape(1, H // 2),
      W3, b3.reshape(1, H // 4), W_out)


def kernel(white_indices, black_indices, W_ft, W1, b1, W2, b2, W3, b3, W_out, b_out):
    tab4 = W_ft.T.reshape(-1, HL)
    ft = _sc_embed(white_indices.astype(jnp.int32).reshape(-1),
                   black_indices.astype(jnp.int32).reshape(-1), tab4)
    x = ft.reshape(B, 2 * H)
    out = _mlp(x, W1, b1, W2, b2, W3, b3, W_out)
    return out[:, 0] + b_out
